# baseline (speedup 1.0000x reference)
import functools

import jax
import jax.numpy as jnp
from jax import lax
from jax.experimental import pallas as pl
from jax.experimental.pallas import tpu as pltpu
from jax.experimental.pallas import tpu_sc as plsc

N = 10000
E = 320000
D = 128
NB = 10
BN = N // NB
NSUB = 16
EPT = E // NSUB
K = 32
NCHUNK = EPT // K
RPT = 624
ZR = 104

_f32 = jnp.float32

_GDN = lax.GatherDimensionNumbers(
    offset_dims=(), collapsed_slice_dims=(0,), start_index_map=(0,))


def _lane_bcast(vec16, l):
    idx = jnp.full((16, 1), l, jnp.int32)
    return lax.gather(vec16, idx, _GDN, (1,),
                      mode=lax.GatherScatterMode.PROMISE_IN_BOUNDS)


NCH_ALL = E // K
CBN = NCH_ALL // NB


def _mm_body(x_ref, xr_ref, w_ref, row_ref, col_ref, ew_ref,
             o1_ref, o2_ref, rc_ref, wrep_ref):
    w = w_ref[...]
    o1_ref[...] = jnp.dot(x_ref[...], w, preferred_element_type=_f32)
    o2_ref[...] = jnp.dot(xr_ref[...], w, preferred_element_type=_f32)
    rc_ref[...] = jnp.concatenate([row_ref[...], col_ref[...]], axis=1)
    sel = (jax.lax.broadcasted_iota(jnp.int32, (K, K * 16), 0) ==
           jax.lax.broadcasted_iota(jnp.int32, (K, K * 16), 1) // 16)
    wrep_ref[...] = jnp.dot(ew_ref[...], sel.astype(_f32),
                            preferred_element_type=_f32)


def _mm(x2, xr2, W, row2d, col2d, ew2d):
    return pl.pallas_call(
        _mm_body,
        grid=(NB,),
        in_specs=[
            pl.BlockSpec((BN, D), lambda i: (i, 0)),
            pl.BlockSpec((BN, D), lambda i: (i, 0)),
            pl.BlockSpec((D, D), lambda i: (0, 0)),
            pl.BlockSpec((CBN, K), lambda i: (i, 0)),
            pl.BlockSpec((CBN, K), lambda i: (i, 0)),
            pl.BlockSpec((CBN, K), lambda i: (i, 0)),
        ],
        out_specs=[
            pl.BlockSpec((BN, D), lambda i: (i, 0)),
            pl.BlockSpec((BN, D), lambda i: (i, 0)),
            pl.BlockSpec((CBN, 2 * K), lambda i: (i, 0)),
            pl.BlockSpec((CBN, 16 * K), lambda i: (i, 0)),
        ],
        out_shape=[
            jax.ShapeDtypeStruct((N, D), _f32),
            jax.ShapeDtypeStruct((N, D), _f32),
            jax.ShapeDtypeStruct((NCH_ALL, 2 * K), jnp.int32),
            jax.ShapeDtypeStruct((NCH_ALL, 16 * K), _f32),
        ],
    )(x2, xr2, W, row2d, col2d, ew2d)


def _spmm_body(seq1, seq2, rc3, w2,
               out1, out2,
               accum,
               rc0, wch0, rc1, wch1, rc2, wch2, rc3b, wch3,
               rows0, rows1, rows2, rows3, zbuf,
               isem0, isem1, isem2, isem3,
               gsem0, gsem1, gsem2, gsem3,
               ssem0, ssem1, ssem2, ssem3):
    c = lax.axis_index("c")
    s = lax.axis_index("s")
    bufs = ((rc0, wch0, rows0, isem0, gsem0, ssem0),
            (rc1, wch1, rows1, isem1, gsem1, ssem1),
            (rc2, wch2, rows2, isem2, gsem2, ssem2),
            (rc3b, wch3, rows3, isem3, gsem3, ssem3))

    def _zrow(i, carry):
        for q in range(D // 16):
            zbuf[i, pl.ds(q * 16, 16)] = jnp.zeros((16,), _f32)
        return carry
    lax.fori_loop(0, ZR, _zrow, 0)
    zbase = pl.multiple_of(s * RPT, 8)
    for p in range(RPT // ZR):
        pltpu.sync_copy(zbuf, accum.at[pl.ds(zbase + p * ZR, ZR)])

    @pl.when(s == NSUB - 1)
    def _():
        pltpu.sync_copy(zbuf.at[pl.ds(0, 16)],
                        accum.at[pl.ds(NSUB * RPT, 16)])

    plsc.subcore_barrier()

    base = s * NCHUNK

    def _issue_idx(k, ch):
        rc, wch, _r, isem, _g, _s = bufs[k]
        pltpu.async_copy(rc3.at[ch], rc, isem)
        pltpu.async_copy(w2.at[ch], wch, isem)

    def _wait_idx(k, ch):
        rc, wch, _r, isem, _g, _s = bufs[k]
        pltpu.make_async_copy(rc3.at[ch], rc, isem).wait()
        pltpu.make_async_copy(w2.at[ch], wch, isem).wait()

    def _edges(table):
        def _issue_gather(k):
            rc, _w, rows, _i, gsem, _s = bufs[k]
            pltpu.async_copy(table.at[rc.at[1]], rows, gsem)

        def _wait_gather(k):
            rc, _w, rows, _i, gsem, _s = bufs[k]
            pltpu.make_async_copy(table.at[rc.at[1]], rows, gsem).wait()

        def _issue_scatter(k):
            rc, _w, rows, _i, _g, ssem = bufs[k]
            pltpu.async_copy(rows, accum.at[rc.at[0]], ssem, add=True)

        def _wait_scatter(k):
            rc, _w, rows, _i, _g, ssem = bufs[k]
            pltpu.make_async_copy(rows, accum.at[rc.at[0]], ssem).wait()

        def _scale(k):
            _rc, wch, rows, _i, _g, _s = bufs[k]

            def _edge(j, c2):
                wv = wch[j, pl.ds(0, 16)]
                for q in range(D // 16):
                    rows[j, pl.ds(q * 16, 16)] = (
                        rows[j, pl.ds(q * 16, 16)] * wv)
                return c2
            lax.fori_loop(0, K, _edge, 0, unroll=4)

        _issue_idx(0, base)
        _issue_idx(1, base + 1)
        _issue_idx(2, base + 2)
        _wait_idx(0, base)
        _issue_gather(0)
        _wait_idx(1, base + 1)
        _issue_gather(1)

        def _quad(g, carry):
            for off in range(4):
                k = off
                ch = base + 4 * g + off
                if off == 0:
                    @pl.when(g >= 1)
                    def _():
                        _wait_scatter(3)
                else:
                    _wait_scatter(off - 1)
                _issue_idx((off + 3) % 4, ch + 3)
                _wait_idx((off + 2) % 4, ch + 2)
                _issue_gather((off + 2) % 4)
                _wait_gather(k)
                _scale(k)
                _issue_scatter(k)
            return carry
        lax.fori_loop(0, NCHUNK // 4 - 1, _quad, 0)

        cb = base + NCHUNK - 5
        _wait_scatter(3)
        _issue_idx(3, cb + 3)
        _wait_idx(2, cb + 2)
        _issue_gather(2)
        _wait_gather(0)
        _scale(0)
        _issue_scatter(0)

        _wait_scatter(0)
        _issue_idx(0, cb + 4)
        _wait_idx(3, cb + 3)
        _issue_gather(3)
        _wait_gather(1)
        _scale(1)
        _issue_scatter(1)

        _wait_scatter(1)
        _wait_idx(0, cb + 4)
        _issue_gather(0)
        _wait_gather(2)
        _scale(2)
        _issue_scatter(2)

        _wait_scatter(2)
        _wait_gather(3)
        _scale(3)
        _issue_scatter(3)

        _wait_scatter(3)
        _wait_gather(0)
        _scale(0)
        _issue_scatter(0)
        _wait_scatter(0)

    @pl.when(c == 0)
    def _():
        _edges(seq1)

    @pl.when(c == 1)
    def _():
        _edges(seq2)

    plsc.subcore_barrier()

    obase = pl.multiple_of(s * RPT, 8)

    @pl.when(c == 0)
    def _():
        pltpu.sync_copy(accum.at[pl.ds(obase, RPT)],
                        out1.at[pl.ds(obase, RPT)])

        @pl.when(s == NSUB - 1)
        def _():
            pltpu.sync_copy(accum.at[pl.ds(NSUB * RPT, 16)],
                            out1.at[pl.ds(NSUB * RPT, 16)])

    @pl.when(c == 1)
    def _():
        pltpu.sync_copy(accum.at[pl.ds(obase, RPT)],
                        out2.at[pl.ds(obase, RPT)])

        @pl.when(s == NSUB - 1)
        def _():
            pltpu.sync_copy(accum.at[pl.ds(NSUB * RPT, 16)],
                            out2.at[pl.ds(NSUB * RPT, 16)])


def _spmm(seq1, seq2, rc2d, wrep2d):
    rc3 = rc2d.reshape(E // K, 2, K)
    w2 = wrep2d.reshape(E // K, K, 16)
    mesh = plsc.VectorSubcoreMesh(core_axis_name="c", subcore_axis_name="s")
    fn = functools.partial(
        pl.kernel,
        mesh=mesh,
        out_type=[
            jax.ShapeDtypeStruct((N, D), _f32),
            jax.ShapeDtypeStruct((N, D), _f32),
        ],
        scratch_types=(
            [pltpu.VMEM_SHARED((N, D), _f32)]
            + [pltpu.VMEM((2, K), jnp.int32),
               pltpu.VMEM((K, 16), _f32)] * 4
            + [pltpu.VMEM((K, D), _f32)] * 4
            + [pltpu.VMEM((ZR, D), _f32)]
            + [pltpu.SemaphoreType.DMA] * 12
        ),
    )(_spmm_body)
    return fn(seq1, seq2, rc3, w2)


def _c1_body(h1p, mskb, bg, a_ref, ssum, msum):
    i = pl.program_id(0)
    a = a_ref[0, 0]
    h1 = h1p[...] + bg[...]
    h1 = jnp.where(h1 >= 0, h1, a * h1)
    m = mskb[...]

    @pl.when(i == 0)
    def _():
        ssum[...] = jnp.zeros_like(ssum)
        msum[...] = jnp.zeros_like(msum)

    ssum[...] += jnp.sum(h1 * m, axis=0, keepdims=True)
    msum[...] += jnp.sum(m).reshape(1, 1)


def _c1(h1p, mskc, bg2, a2):
    return pl.pallas_call(
        _c1_body,
        grid=(NB,),
        in_specs=[
            pl.BlockSpec((BN, D), lambda i: (i, 0)),
            pl.BlockSpec((BN, 1), lambda i: (i, 0)),
            pl.BlockSpec((1, D), lambda i: (0, 0)),
            pl.BlockSpec((1, 1), lambda i: (0, 0)),
        ],
        out_specs=[
            pl.BlockSpec((1, D), lambda i: (0, 0)),
            pl.BlockSpec((1, 1), lambda i: (0, 0)),
        ],
        out_shape=[
            jax.ShapeDtypeStruct((1, D), _f32),
            jax.ShapeDtypeStruct((1, 1), _f32),
        ],
    )(h1p, mskc, bg2, a2)


def _c2_body(h1p, h2p, fb, frb, ssum, msum, bg, a_ref, wE, wI, wJ,
             sb1, sb2, bvec, e1, i1, j1):
    sv = jax.nn.sigmoid(ssum[...] / msum[0, 0])
    vE = jnp.sum(wE[...] * sv, axis=1)[None, :]
    bE = bvec[0, 0]
    bI = bvec[0, 1]
    bJ = bvec[0, 2]
    a = a_ref[0, 0]
    b = bg[...]
    h1v = h1p[...] + b
    h1v = jnp.where(h1v >= 0, h1v, a * h1v)
    h2v = h2p[...] + b
    h2v = jnp.where(h2v >= 0, h2v, a * h2v)
    fv = fb[...]
    frv = frb[...]
    s1 = sb1[...]
    s2 = sb2[...]
    e1[0] = jnp.sum(h1v * vE, axis=1, keepdims=True) + bE + s1
    e1[1] = jnp.sum(h2v * vE, axis=1, keepdims=True) + bE + s2
    P = jnp.dot(h1v, wI[...], preferred_element_type=_f32)
    i1[0] = jnp.sum(P * fv, axis=1, keepdims=True) + bI + s1
    i1[1] = jnp.sum(P * frv, axis=1, keepdims=True) + bI + s2
    Q = jnp.dot(h1v * sv, wJ[...], preferred_element_type=_f32)
    j1[0] = jnp.sum(Q * fv, axis=1, keepdims=True) + bJ + s1
    j1[1] = jnp.sum(Q * frv, axis=1, keepdims=True) + bJ + s2


def _c2(h1p, h2p, f2, fr2, ssum, msum, bg2, a2, W_E, W_I, W_J,
        sb1, sb2, bvec):
    vec = lambda: pl.BlockSpec((BN, 1), lambda i: (i, 0))
    blk = lambda: pl.BlockSpec((BN, D), lambda i: (i, 0))
    fix = lambda r, c: pl.BlockSpec((r, c), lambda i: (0, 0))
    return pl.pallas_call(
        _c2_body,
        grid=(NB,),
        in_specs=[
            blk(), blk(), blk(), blk(),
            fix(1, D), fix(1, 1), fix(1, D), fix(1, 1),
            fix(D, D), fix(D, D), fix(D, D),
            vec(), vec(), fix(1, 3),
        ],
        out_specs=[pl.BlockSpec((2, BN, 1), lambda i: (0, i, 0))
                   for _ in range(3)],
        out_shape=[jax.ShapeDtypeStruct((2, N, 1), _f32) for _ in range(3)],
    )(h1p, h2p, f2, fr2, ssum, msum, bg2, a2, W_E, W_I, W_J, sb1, sb2, bvec)


def kernel(x, x_r, f, f_r, edge_index, edge_weight, msk, samp_bias1,
           samp_bias2, sparse, W_gcn, b_gcn, prelu_a, W_E, b_E, W_I, b_I,
           W_J, b_J):
    x2 = x[0]
    xr2 = x_r[0]
    f2 = f[0]
    fr2 = f_r[0]
    row2d = edge_index[0].reshape(E // K, K)
    col2d = edge_index[1].reshape(E // K, K)
    ew2d = edge_weight.reshape(E // K, K)

    seq1, seq2, rc2d, wrep2d = _mm(x2, xr2, W_gcn, row2d, col2d, ew2d)
    h1p, h2p = _spmm(seq1, seq2, rc2d, wrep2d)

    bg2 = b_gcn.reshape(1, D)
    a2 = prelu_a.reshape(1, 1)
    mskc = msk.reshape(N, 1)
    ssum, msum = _c1(h1p, mskc, bg2, a2)

    bvec = jnp.stack([b_E, b_I, b_J]).reshape(1, 3)
    eo, io, jo = _c2(h1p, h2p, f2, fr2, ssum, msum, bg2, a2, W_E, W_I, W_J,
                     samp_bias1.reshape(N, 1),
                     samp_bias2.reshape(N, 1), bvec)

    return (eo.reshape(1, 2 * N), io.reshape(1, 2 * N), jo.reshape(1, 2 * N))


# speedup vs baseline: 1.2364x; 1.2364x over previous
import functools

import jax
import jax.numpy as jnp
from jax import lax
from jax.experimental import pallas as pl
from jax.experimental.pallas import tpu as pltpu
from jax.experimental.pallas import tpu_sc as plsc

N = 10000
E = 320000
D = 128
NB = 10
BN = N // NB
NSUB = 16
EPT = E // NSUB
K = 80
NCHUNK = EPT // K
RPT = 624
ZR = 8

_f32 = jnp.float32

_GDN = lax.GatherDimensionNumbers(
    offset_dims=(), collapsed_slice_dims=(0,), start_index_map=(0,))


def _lane_bcast(vec16, l):
    idx = jnp.full((16, 1), l, jnp.int32)
    return lax.gather(vec16, idx, _GDN, (1,),
                      mode=lax.GatherScatterMode.PROMISE_IN_BOUNDS)


NCH_ALL = E // K
CBN = NCH_ALL // NB


def _mm_body(x_ref, xr_ref, w_ref, row_ref, col_ref, ew_ref,
             o1_ref, o2_ref, rc_ref, wrep_ref):
    w = w_ref[...]
    o1_ref[...] = jnp.dot(x_ref[...], w, preferred_element_type=_f32)
    o2_ref[...] = jnp.dot(xr_ref[...], w, preferred_element_type=_f32)
    rc_ref[...] = jnp.concatenate([row_ref[...], col_ref[...]], axis=1)
    sel = (jax.lax.broadcasted_iota(jnp.int32, (K, K * 16), 0) ==
           jax.lax.broadcasted_iota(jnp.int32, (K, K * 16), 1) // 16)
    wrep_ref[...] = jnp.dot(ew_ref[...], sel.astype(_f32),
                            preferred_element_type=_f32)


def _mm(x2, xr2, W, row2d, col2d, ew2d):
    return pl.pallas_call(
        _mm_body,
        grid=(NB,),
        in_specs=[
            pl.BlockSpec((BN, D), lambda i: (i, 0)),
            pl.BlockSpec((BN, D), lambda i: (i, 0)),
            pl.BlockSpec((D, D), lambda i: (0, 0)),
            pl.BlockSpec((CBN, K), lambda i: (i, 0)),
            pl.BlockSpec((CBN, K), lambda i: (i, 0)),
            pl.BlockSpec((CBN, K), lambda i: (i, 0)),
        ],
        out_specs=[
            pl.BlockSpec((BN, D), lambda i: (i, 0)),
            pl.BlockSpec((BN, D), lambda i: (i, 0)),
            pl.BlockSpec((CBN, 2 * K), lambda i: (i, 0)),
            pl.BlockSpec((CBN, 16 * K), lambda i: (i, 0)),
        ],
        out_shape=[
            jax.ShapeDtypeStruct((N, D), _f32),
            jax.ShapeDtypeStruct((N, D), _f32),
            jax.ShapeDtypeStruct((NCH_ALL, 2 * K), jnp.int32),
            jax.ShapeDtypeStruct((NCH_ALL, 16 * K), _f32),
        ],
    )(x2, xr2, W, row2d, col2d, ew2d)


def _spmm_body(seq1, seq2, rc3, w2,
               out1, out2,
               accum,
               rc0, rc1, rc2, rc3b, wch0, wch1,
               rows0, rows1, zbuf,
               isem0, isem1, isem2, isem3, wsem0, wsem1,
               gsem0, gsem1, ssem0, ssem1):
    c = lax.axis_index("c")
    s = lax.axis_index("s")
    idxb = ((rc0, isem0), (rc1, isem1), (rc2, isem2), (rc3b, isem3))
    wchb = ((wch0, wsem0), (wch1, wsem1))
    rowb = ((rows0, gsem0, ssem0),
            (rows1, gsem1, ssem1))

    def _zrow(i, carry):
        for q in range(D // 16):
            zbuf[i, pl.ds(q * 16, 16)] = jnp.zeros((16,), _f32)
        return carry
    lax.fori_loop(0, ZR, _zrow, 0)
    zbase = pl.multiple_of(s * RPT, 8)
    for p in range(RPT // ZR):
        pltpu.sync_copy(zbuf, accum.at[pl.ds(zbase + p * ZR, ZR)])

    @pl.when(s == NSUB - 1)
    def _():
        pltpu.sync_copy(zbuf.at[pl.ds(0, 16)],
                        accum.at[pl.ds(NSUB * RPT, 16)])

    plsc.subcore_barrier()

    base = s * NCHUNK

    def _issue_idx(k, ch):
        rc, isem = idxb[k]
        pltpu.async_copy(rc3.at[ch], rc, isem)

    def _wait_idx(k, ch):
        rc, isem = idxb[k]
        pltpu.make_async_copy(rc3.at[ch], rc, isem).wait()

    def _issue_wch(m, ch):
        wch, wsem = wchb[m]
        pltpu.async_copy(w2.at[ch], wch, wsem)

    def _wait_wch(m, ch):
        wch, wsem = wchb[m]
        pltpu.make_async_copy(w2.at[ch], wch, wsem).wait()

    def _edges(table):
        def _issue_gather(m, k):
            rows, gsem, _s = rowb[m]
            pltpu.async_copy(table.at[idxb[k][0].at[1]], rows, gsem)

        def _wait_gather(m, k):
            rows, gsem, _s = rowb[m]
            pltpu.make_async_copy(table.at[idxb[k][0].at[1]], rows, gsem).wait()

        def _issue_scatter(m, k):
            rows, _g, ssem = rowb[m]
            pltpu.async_copy(rows, accum.at[idxb[k][0].at[0]], ssem, add=True)

        def _wait_scatter(m, k):
            rows, _g, ssem = rowb[m]
            pltpu.make_async_copy(rows, accum.at[idxb[k][0].at[0]], ssem).wait()

        def _scale(m, k):
            rows = rowb[m][0]
            wch = wchb[m][0]

            def _edge(j, c2):
                wv = wch[j, pl.ds(0, 16)]
                for q in range(D // 16):
                    rows[j, pl.ds(q * 16, 16)] = (
                        rows[j, pl.ds(q * 16, 16)] * wv)
                return c2
            lax.fori_loop(0, K, _edge, 0, unroll=4)

        _issue_idx(0, base)
        _issue_wch(0, base)
        _issue_wch(1, base + 1)
        _wait_idx(0, base)
        _issue_gather(0, 0)
        _issue_idx(1, base + 1)

        def _quad(g, carry):
            for off in range(4):
                m, mn = off % 2, (off + 1) % 2
                k, kn, ki = off, (off + 1) % 4, (off + 2) % 4
                ch = base + 4 * g + off
                _wait_idx(kn, ch + 1)
                if off == 0:
                    @pl.when(g >= 1)
                    def _():
                        _wait_scatter(mn, (off + 3) % 4)
                else:
                    _wait_scatter(mn, (off + 3) % 4)
                _issue_idx(ki, ch + 2)
                _issue_gather(mn, kn)
                _wait_gather(m, k)
                _wait_wch(m, ch)
                _scale(m, k)
                _issue_scatter(m, k)
                _issue_wch(m, ch + 2)
            return carry
        lax.fori_loop(0, NCHUNK // 4 - 1, _quad, 0)

        cb = base + NCHUNK - 6
        _wait_idx(1, cb + 1)
        _wait_scatter(1, 3)
        _issue_idx(2, cb + 2)
        _issue_gather(1, 1)
        _wait_gather(0, 0)
        _wait_wch(0, cb)
        _scale(0, 0)
        _issue_scatter(0, 0)
        _issue_wch(0, cb + 2)

        _wait_idx(2, cb + 2)
        _wait_scatter(0, 0)
        _issue_idx(3, cb + 3)
        _issue_gather(0, 2)
        _wait_gather(1, 1)
        _wait_wch(1, cb + 1)
        _scale(1, 1)
        _issue_scatter(1, 1)
        _issue_wch(1, cb + 3)

        _wait_idx(3, cb + 3)
        _wait_scatter(1, 1)
        _issue_idx(0, cb + 4)
        _issue_gather(1, 3)
        _wait_gather(0, 2)
        _wait_wch(0, cb + 2)
        _scale(0, 2)
        _issue_scatter(0, 2)
        _issue_wch(0, cb + 4)

        _wait_idx(0, cb + 4)
        _wait_scatter(0, 2)
        _issue_idx(1, cb + 5)
        _issue_gather(0, 0)
        _wait_gather(1, 3)
        _wait_wch(1, cb + 3)
        _scale(1, 3)
        _issue_scatter(1, 3)
        _issue_wch(1, cb + 5)

        _wait_idx(1, cb + 5)
        _wait_scatter(1, 3)
        _issue_gather(1, 1)
        _wait_gather(0, 0)
        _wait_wch(0, cb + 4)
        _scale(0, 0)
        _issue_scatter(0, 0)

        _wait_scatter(0, 0)
        _wait_gather(1, 1)
        _wait_wch(1, cb + 5)
        _scale(1, 1)
        _issue_scatter(1, 1)
        _wait_scatter(1, 1)

    @pl.when(c == 0)
    def _():
        _edges(seq1)

    @pl.when(c == 1)
    def _():
        _edges(seq2)

    plsc.subcore_barrier()

    obase = pl.multiple_of(s * RPT, 8)

    @pl.when(c == 0)
    def _():
        pltpu.sync_copy(accum.at[pl.ds(obase, RPT)],
                        out1.at[pl.ds(obase, RPT)])

        @pl.when(s == NSUB - 1)
        def _():
            pltpu.sync_copy(accum.at[pl.ds(NSUB * RPT, 16)],
                            out1.at[pl.ds(NSUB * RPT, 16)])

    @pl.when(c == 1)
    def _():
        pltpu.sync_copy(accum.at[pl.ds(obase, RPT)],
                        out2.at[pl.ds(obase, RPT)])

        @pl.when(s == NSUB - 1)
        def _():
            pltpu.sync_copy(accum.at[pl.ds(NSUB * RPT, 16)],
                            out2.at[pl.ds(NSUB * RPT, 16)])


def _spmm(seq1, seq2, rc2d, wrep2d):
    rc3 = rc2d.reshape(E // K, 2, K)
    w2 = wrep2d.reshape(E // K, K, 16)
    mesh = plsc.VectorSubcoreMesh(core_axis_name="c", subcore_axis_name="s")
    fn = functools.partial(
        pl.kernel,
        mesh=mesh,
        out_type=[
            jax.ShapeDtypeStruct((N, D), _f32),
            jax.ShapeDtypeStruct((N, D), _f32),
        ],
        scratch_types=(
            [pltpu.VMEM_SHARED((N, D), _f32)]
            + [pltpu.VMEM((2, K), jnp.int32)] * 4
            + [pltpu.VMEM((K, 16), _f32)] * 2
            + [pltpu.VMEM((K, D), _f32),
               pltpu.VMEM((K, D), _f32),
               pltpu.VMEM((ZR, D), _f32)]
            + [pltpu.SemaphoreType.DMA] * 10
        ),
    )(_spmm_body)
    return fn(seq1, seq2, rc3, w2)


def _c1_body(h1p, mskb, bg, a_ref, ssum, msum):
    i = pl.program_id(0)
    a = a_ref[0, 0]
    h1 = h1p[...] + bg[...]
    h1 = jnp.where(h1 >= 0, h1, a * h1)
    m = mskb[...]

    @pl.when(i == 0)
    def _():
        ssum[...] = jnp.zeros_like(ssum)
        msum[...] = jnp.zeros_like(msum)

    ssum[...] += jnp.sum(h1 * m, axis=0, keepdims=True)
    msum[...] += jnp.sum(m).reshape(1, 1)


def _c1(h1p, mskc, bg2, a2):
    return pl.pallas_call(
        _c1_body,
        grid=(NB,),
        in_specs=[
            pl.BlockSpec((BN, D), lambda i: (i, 0)),
            pl.BlockSpec((BN, 1), lambda i: (i, 0)),
            pl.BlockSpec((1, D), lambda i: (0, 0)),
            pl.BlockSpec((1, 1), lambda i: (0, 0)),
        ],
        out_specs=[
            pl.BlockSpec((1, D), lambda i: (0, 0)),
            pl.BlockSpec((1, 1), lambda i: (0, 0)),
        ],
        out_shape=[
            jax.ShapeDtypeStruct((1, D), _f32),
            jax.ShapeDtypeStruct((1, 1), _f32),
        ],
    )(h1p, mskc, bg2, a2)


def _c2_body(h1p, h2p, fb, frb, ssum, msum, bg, a_ref, wE, wI, wJ,
             sb1, sb2, bvec, e1, i1, j1):
    sv = jax.nn.sigmoid(ssum[...] / msum[0, 0])
    vE = jnp.sum(wE[...] * sv, axis=1)[None, :]
    bE = bvec[0, 0]
    bI = bvec[0, 1]
    bJ = bvec[0, 2]
    a = a_ref[0, 0]
    b = bg[...]
    h1v = h1p[...] + b
    h1v = jnp.where(h1v >= 0, h1v, a * h1v)
    h2v = h2p[...] + b
    h2v = jnp.where(h2v >= 0, h2v, a * h2v)
    fv = fb[...]
    frv = frb[...]
    s1 = sb1[...]
    s2 = sb2[...]
    e1[0] = jnp.sum(h1v * vE, axis=1, keepdims=True) + bE + s1
    e1[1] = jnp.sum(h2v * vE, axis=1, keepdims=True) + bE + s2
    P = jnp.dot(h1v, wI[...], preferred_element_type=_f32)
    i1[0] = jnp.sum(P * fv, axis=1, keepdims=True) + bI + s1
    i1[1] = jnp.sum(P * frv, axis=1, keepdims=True) + bI + s2
    Q = jnp.dot(h1v * sv, wJ[...], preferred_element_type=_f32)
    j1[0] = jnp.sum(Q * fv, axis=1, keepdims=True) + bJ + s1
    j1[1] = jnp.sum(Q * frv, axis=1, keepdims=True) + bJ + s2


def _c2(h1p, h2p, f2, fr2, ssum, msum, bg2, a2, W_E, W_I, W_J,
        sb1, sb2, bvec):
    vec = lambda: pl.BlockSpec((BN, 1), lambda i: (i, 0))
    blk = lambda: pl.BlockSpec((BN, D), lambda i: (i, 0))
    fix = lambda r, c: pl.BlockSpec((r, c), lambda i: (0, 0))
    return pl.pallas_call(
        _c2_body,
        grid=(NB,),
        in_specs=[
            blk(), blk(), blk(), blk(),
            fix(1, D), fix(1, 1), fix(1, D), fix(1, 1),
            fix(D, D), fix(D, D), fix(D, D),
            vec(), vec(), fix(1, 3),
        ],
        out_specs=[pl.BlockSpec((2, BN, 1), lambda i: (0, i, 0))
                   for _ in range(3)],
        out_shape=[jax.ShapeDtypeStruct((2, N, 1), _f32) for _ in range(3)],
    )(h1p, h2p, f2, fr2, ssum, msum, bg2, a2, W_E, W_I, W_J, sb1, sb2, bvec)


def kernel(x, x_r, f, f_r, edge_index, edge_weight, msk, samp_bias1,
           samp_bias2, sparse, W_gcn, b_gcn, prelu_a, W_E, b_E, W_I, b_I,
           W_J, b_J):
    x2 = x[0]
    xr2 = x_r[0]
    f2 = f[0]
    fr2 = f_r[0]
    row2d = edge_index[0].reshape(E // K, K)
    col2d = edge_index[1].reshape(E // K, K)
    ew2d = edge_weight.reshape(E // K, K)

    seq1, seq2, rc2d, wrep2d = _mm(x2, xr2, W_gcn, row2d, col2d, ew2d)
    h1p, h2p = _spmm(seq1, seq2, rc2d, wrep2d)

    bg2 = b_gcn.reshape(1, D)
    a2 = prelu_a.reshape(1, 1)
    mskc = msk.reshape(N, 1)
    ssum, msum = _c1(h1p, mskc, bg2, a2)

    bvec = jnp.stack([b_E, b_I, b_J]).reshape(1, 3)
    eo, io, jo = _c2(h1p, h2p, f2, fr2, ssum, msum, bg2, a2, W_E, W_I, W_J,
                     samp_bias1.reshape(N, 1),
                     samp_bias2.reshape(N, 1), bvec)

    return (eo.reshape(1, 2 * N), io.reshape(1, 2 * N), jo.reshape(1, 2 * N))
